# Initial kernel scaffold; baseline (speedup 1.0000x reference)
import jax
import jax.numpy as jnp
from jax.experimental import pallas as pl


def kernel(x, edge_index, W1, b1, W2, b2):
    raise NotImplementedError("write your pallas kernel here")


# speedup vs baseline: 17.7071x; 17.7071x over previous
import functools

import jax
import jax.numpy as jnp
from jax import lax
from jax.experimental import pallas as pl
from jax.experimental.pallas import tpu as pltpu
from jax.experimental.pallas import tpu_sc as plsc

_N = 10000
_E = 320000
_D_IN, _D_HID, _D_OUT = 128, 64, 32

_NC, _NS = 2, 16
_NW = _NC * _NS
_C = 128
_K = 80
_EPAD = _NW * _K * _C
_ACC = 10240
_RPT = _ACC // _NS
_DUMP = _N

_BN = 2000


def _mesh():
    return plsc.VectorSubcoreMesh(core_axis_name="c", subcore_axis_name="s")


@functools.partial(
    pl.kernel,
    out_type=jax.ShapeDtypeStruct((_NC, _ACC, 16), jnp.float32),
    mesh=_mesh(),
    compiler_params=pltpu.CompilerParams(use_tc_tiling_on_sc=False),
    scratch_types=[
        pltpu.VMEM((_K, _C), jnp.int32),
        pltpu.VMEM((_C, 16), jnp.float32),
        pltpu.VMEM((_RPT, 16), jnp.float32),
        pltpu.VMEM_SHARED((_ACC, 16), jnp.float32),
    ],
)
def _deg_kernel(dst_hbm, out_hbm, dst_v, ones_v, stage_v, acc_sh):
    cid = lax.axis_index("c")
    sid = lax.axis_index("s")
    wid = sid * _NC + cid

    def fill_ones(i, c):
        ones_v[i, :] = jnp.ones((16,), jnp.float32)
        return c

    lax.fori_loop(0, _C, fill_ones, 0)

    def fill_zero(i, c):
        stage_v[i, :] = jnp.zeros((16,), jnp.float32)
        return c

    lax.fori_loop(0, _RPT, fill_zero, 0)

    pltpu.sync_copy(stage_v, acc_sh.at[pl.ds(sid * _RPT, _RPT)])
    plsc.subcore_barrier()

    pltpu.sync_copy(dst_hbm.at[wid], dst_v)

    def chunk(j, c):
        pltpu.sync_copy(ones_v, acc_sh.at[dst_v.at[j]], add=True)
        return c

    lax.fori_loop(0, _K, chunk, 0)

    plsc.subcore_barrier()
    pltpu.sync_copy(acc_sh.at[pl.ds(sid * _RPT, _RPT)], stage_v)
    pltpu.sync_copy(stage_v, out_hbm.at[cid, pl.ds(sid * _RPT, _RPT)])


def _make_agg(D):
    @functools.partial(
        pl.kernel,
        out_type=jax.ShapeDtypeStruct((_NC, _ACC, D), jnp.float32),
        mesh=_mesh(),
        compiler_params=pltpu.CompilerParams(use_tc_tiling_on_sc=False),
        scratch_types=[
            pltpu.VMEM((_K, _C), jnp.int32),
            pltpu.VMEM((_K, _C), jnp.int32),
            pltpu.VMEM((_C, D), jnp.float32),
            pltpu.VMEM((_C, D), jnp.float32),
            pltpu.VMEM_SHARED((_ACC, D), jnp.float32),
            pltpu.SemaphoreType.DMA,
        ],
    )
    def _agg(s_hbm, src_hbm, dst_hbm, out_hbm,
             src_v, dst_v, rows_v, zb, acc_sh, sem):
        cid = lax.axis_index("c")
        sid = lax.axis_index("s")
        wid = sid * _NC + cid
        nz = D // 16

        def fill_zero(i, c):
            for l in range(nz):
                zb[i, pl.ds(l * 16, 16)] = jnp.zeros((16,), jnp.float32)
            return c

        lax.fori_loop(0, _C, fill_zero, 0)

        for q in range(_RPT // _C):
            pltpu.sync_copy(zb, acc_sh.at[pl.ds(sid * _RPT + q * _C, _C)])
        plsc.subcore_barrier()

        pltpu.sync_copy(src_hbm.at[wid], src_v)
        pltpu.sync_copy(dst_hbm.at[wid], dst_v)

        def chunk(j, c):
            pltpu.async_copy(s_hbm.at[src_v.at[j]], rows_v, sem).wait()
            pltpu.sync_copy(rows_v, acc_sh.at[dst_v.at[j]], add=True)
            return c

        lax.fori_loop(0, _K, chunk, 0)

        plsc.subcore_barrier()
        for q in range(_RPT // _C):
            pltpu.sync_copy(acc_sh.at[pl.ds(sid * _RPT + q * _C, _C)], zb)
            pltpu.sync_copy(zb, out_hbm.at[cid, pl.ds(sid * _RPT + q * _C, _C)])

    return _agg


def _tc1_body(x_ref, w1_ref, degp_ref, h1_ref, s1_ref):
    deg = degp_ref[0, :, 0:1] + degp_ref[1, :, 0:1] + 1.0
    dis = lax.rsqrt(deg)
    h = jnp.dot(x_ref[...], w1_ref[...], preferred_element_type=jnp.float32)
    h1_ref[...] = h
    s1_ref[...] = h * dis


def _tc2_body(degp_ref, p1_ref, h1_ref, b1_ref, w2_ref, h2_ref, s2_ref):
    deg = degp_ref[0, :, 0:1] + degp_ref[1, :, 0:1] + 1.0
    dis = lax.rsqrt(deg)
    agg = p1_ref[0] + p1_ref[1]
    x1 = dis * agg + (dis * dis) * h1_ref[...] + b1_ref[...]
    r = jnp.maximum(x1, 0.0)
    h2 = jnp.dot(r, w2_ref[...], preferred_element_type=jnp.float32)
    h2_ref[...] = h2
    s2_ref[...] = h2 * dis


def _tc3_body(degp_ref, p2_ref, h2_ref, b2_ref, out_ref):
    deg = degp_ref[0, :, 0:1] + degp_ref[1, :, 0:1] + 1.0
    dis = lax.rsqrt(deg)
    agg = p2_ref[0] + p2_ref[1]
    out_ref[...] = dis * agg + (dis * dis) * h2_ref[...] + b2_ref[...]


def _row_spec(d):
    return pl.BlockSpec((_BN, d), lambda i: (i, 0))


def _part_spec(d):
    return pl.BlockSpec((2, _BN, d), lambda i: (0, i, 0))


def _full_spec(shape):
    return pl.BlockSpec(shape, lambda i: tuple(0 for _ in shape))


_GRID = _N // _BN


def kernel(x, edge_index, W1, b1, W2, b2):
    src = edge_index[0].astype(jnp.int32)
    dst = edge_index[1].astype(jnp.int32)
    pad = _EPAD - _E
    src_p = jnp.concatenate([src, jnp.zeros((pad,), jnp.int32)]).reshape(_NW, _K, _C)
    dst_p = jnp.concatenate([dst, jnp.full((pad,), _DUMP, jnp.int32)]).reshape(_NW, _K, _C)

    degp = _deg_kernel(dst_p)

    h1, s1 = pl.pallas_call(
        _tc1_body,
        grid=(_GRID,),
        in_specs=[_row_spec(_D_IN), _full_spec((_D_IN, _D_HID)), _part_spec(16)],
        out_specs=[_row_spec(_D_HID), _row_spec(_D_HID)],
        out_shape=[jax.ShapeDtypeStruct((_N, _D_HID), jnp.float32)] * 2,
    )(x, W1, degp)

    p1 = _make_agg(_D_HID)(s1, src_p, dst_p)

    h2, s2 = pl.pallas_call(
        _tc2_body,
        grid=(_GRID,),
        in_specs=[_part_spec(16), _part_spec(_D_HID), _row_spec(_D_HID),
                  _full_spec((1, _D_HID)), _full_spec((_D_HID, _D_OUT))],
        out_specs=[_row_spec(_D_OUT), _row_spec(_D_OUT)],
        out_shape=[jax.ShapeDtypeStruct((_N, _D_OUT), jnp.float32)] * 2,
    )(degp, p1, h1, b1.reshape(1, _D_HID), W2)

    p2 = _make_agg(_D_OUT)(s2, src_p, dst_p)

    out = pl.pallas_call(
        _tc3_body,
        grid=(_GRID,),
        in_specs=[_part_spec(16), _part_spec(_D_OUT), _row_spec(_D_OUT),
                  _full_spec((1, _D_OUT))],
        out_specs=_row_spec(_D_OUT),
        out_shape=jax.ShapeDtypeStruct((_N, _D_OUT), jnp.float32),
    )(degp, p2, h2, b2.reshape(1, _D_OUT))

    return out


# speedup vs baseline: 21.3514x; 1.2058x over previous
import functools

import jax
import jax.numpy as jnp
from jax import lax
from jax.experimental import pallas as pl
from jax.experimental.pallas import tpu as pltpu
from jax.experimental.pallas import tpu_sc as plsc

_N = 10000
_E = 320000
_D_IN, _D_HID, _D_OUT = 128, 64, 32

_NC, _NS = 2, 16
_NW = _NC * _NS
_C = 128
_K = 80
_EPAD = _NW * _K * _C
_ACC = 10240
_RPT = _ACC // _NS
_DUMP = _N

_BN = 2000


def _mesh():
    return plsc.VectorSubcoreMesh(core_axis_name="c", subcore_axis_name="s")


@functools.partial(
    pl.kernel,
    out_type=jax.ShapeDtypeStruct((_NC, _ACC, 16), jnp.float32),
    mesh=_mesh(),
    compiler_params=pltpu.CompilerParams(use_tc_tiling_on_sc=False),
    scratch_types=[
        pltpu.VMEM((_K, _C), jnp.int32),
        pltpu.VMEM((_C, 16), jnp.float32),
        pltpu.VMEM((_RPT, 16), jnp.float32),
        pltpu.VMEM_SHARED((_ACC, 16), jnp.float32),
    ],
)
def _deg_kernel(dst_hbm, out_hbm, dst_v, ones_v, stage_v, acc_sh):
    cid = lax.axis_index("c")
    sid = lax.axis_index("s")
    wid = sid * _NC + cid

    def fill_ones(i, c):
        ones_v[i, :] = jnp.ones((16,), jnp.float32)
        return c

    lax.fori_loop(0, _C, fill_ones, 0)

    def fill_zero(i, c):
        stage_v[i, :] = jnp.zeros((16,), jnp.float32)
        return c

    lax.fori_loop(0, _RPT, fill_zero, 0)

    pltpu.sync_copy(stage_v, acc_sh.at[pl.ds(sid * _RPT, _RPT)])
    plsc.subcore_barrier()

    pltpu.sync_copy(dst_hbm.at[wid], dst_v)

    def chunk(j, c):
        pltpu.sync_copy(ones_v, acc_sh.at[dst_v.at[j]], add=True)
        return c

    lax.fori_loop(0, _K, chunk, 0)

    plsc.subcore_barrier()
    pltpu.sync_copy(acc_sh.at[pl.ds(sid * _RPT, _RPT)], stage_v)
    pltpu.sync_copy(stage_v, out_hbm.at[cid, pl.ds(sid * _RPT, _RPT)])


def _make_agg(D):
    @functools.partial(
        pl.kernel,
        out_type=jax.ShapeDtypeStruct((_NC, _ACC, D), jnp.float32),
        mesh=_mesh(),
        compiler_params=pltpu.CompilerParams(use_tc_tiling_on_sc=False),
        scratch_types=[
            pltpu.VMEM((_K, _C), jnp.int32),
            pltpu.VMEM((_K, _C), jnp.int32),
            pltpu.VMEM((_C, D), jnp.float32),
            pltpu.VMEM((_C, D), jnp.float32),
            pltpu.VMEM((_C, D), jnp.float32),
            pltpu.VMEM((_C, D), jnp.float32),
            pltpu.VMEM((_C, D), jnp.float32),
            pltpu.VMEM_SHARED((_ACC, D), jnp.float32),
            pltpu.SemaphoreType.DMA,
            pltpu.SemaphoreType.DMA,
            pltpu.SemaphoreType.DMA,
            pltpu.SemaphoreType.DMA,
        ],
    )
    def _agg(s_hbm, src_hbm, dst_hbm, out_hbm,
             src_v, dst_v, r0, r1, r2, r3, zb, acc_sh, g0, g1, g2, g3):
        cid = lax.axis_index("c")
        sid = lax.axis_index("s")
        wid = sid * _NC + cid
        nz = D // 16

        def fill_zero(i, c):
            for l in range(nz):
                zb[i, pl.ds(l * 16, 16)] = jnp.zeros((16,), jnp.float32)
            return c

        lax.fori_loop(0, _C, fill_zero, 0)

        for q in range(_RPT // _C):
            pltpu.sync_copy(zb, acc_sh.at[pl.ds(sid * _RPT + q * _C, _C)])
        plsc.subcore_barrier()

        pltpu.sync_copy(src_hbm.at[wid], src_v)
        pltpu.sync_copy(dst_hbm.at[wid], dst_v)

        bufs = (r0, r1, r2, r3)
        sems = (g0, g1, g2, g3)
        nb = 4

        for l in range(nb):
            pltpu.async_copy(s_hbm.at[src_v.at[l]], bufs[l], sems[l])

        def step(j4, c):
            for l in range(nb):
                jc = j4 * nb + l
                pltpu.make_async_copy(
                    s_hbm.at[src_v.at[jc]], bufs[l], sems[l]).wait()
                pltpu.sync_copy(bufs[l], acc_sh.at[dst_v.at[jc]], add=True)
                pltpu.async_copy(
                    s_hbm.at[src_v.at[jc + nb]], bufs[l], sems[l])
            return c

        lax.fori_loop(0, _K // nb - 1, step, 0)

        for l in range(nb):
            jc = _K - nb + l
            pltpu.make_async_copy(
                s_hbm.at[src_v.at[jc]], bufs[l], sems[l]).wait()
            pltpu.sync_copy(bufs[l], acc_sh.at[dst_v.at[jc]], add=True)

        plsc.subcore_barrier()
        for q in range(_RPT // _C):
            pltpu.sync_copy(acc_sh.at[pl.ds(sid * _RPT + q * _C, _C)], zb)
            pltpu.sync_copy(zb, out_hbm.at[cid, pl.ds(sid * _RPT + q * _C, _C)])

    return _agg


def _tc1_body(x_ref, w1_ref, degp_ref, h1_ref, s1_ref):
    deg = degp_ref[0, :, 0:1] + degp_ref[1, :, 0:1] + 1.0
    dis = lax.rsqrt(deg)
    h = jnp.dot(x_ref[...], w1_ref[...], preferred_element_type=jnp.float32)
    h1_ref[...] = h
    s1_ref[...] = h * dis


def _tc2_body(degp_ref, p1_ref, h1_ref, b1_ref, w2_ref, h2_ref, s2_ref):
    deg = degp_ref[0, :, 0:1] + degp_ref[1, :, 0:1] + 1.0
    dis = lax.rsqrt(deg)
    agg = p1_ref[0] + p1_ref[1]
    x1 = dis * agg + (dis * dis) * h1_ref[...] + b1_ref[...]
    r = jnp.maximum(x1, 0.0)
    h2 = jnp.dot(r, w2_ref[...], preferred_element_type=jnp.float32)
    h2_ref[...] = h2
    s2_ref[...] = h2 * dis


def _tc3_body(degp_ref, p2_ref, h2_ref, b2_ref, out_ref):
    deg = degp_ref[0, :, 0:1] + degp_ref[1, :, 0:1] + 1.0
    dis = lax.rsqrt(deg)
    agg = p2_ref[0] + p2_ref[1]
    out_ref[...] = dis * agg + (dis * dis) * h2_ref[...] + b2_ref[...]


def _row_spec(d):
    return pl.BlockSpec((_BN, d), lambda i: (i, 0))


def _part_spec(d):
    return pl.BlockSpec((2, _BN, d), lambda i: (0, i, 0))


def _full_spec(shape):
    return pl.BlockSpec(shape, lambda i: tuple(0 for _ in shape))


_GRID = _N // _BN


def kernel(x, edge_index, W1, b1, W2, b2):
    src = edge_index[0].astype(jnp.int32)
    dst = edge_index[1].astype(jnp.int32)
    pad = _EPAD - _E
    src_p = jnp.concatenate([src, jnp.zeros((pad,), jnp.int32)]).reshape(_NW, _K, _C)
    dst_p = jnp.concatenate([dst, jnp.full((pad,), _DUMP, jnp.int32)]).reshape(_NW, _K, _C)

    degp = _deg_kernel(dst_p)

    h1, s1 = pl.pallas_call(
        _tc1_body,
        grid=(_GRID,),
        in_specs=[_row_spec(_D_IN), _full_spec((_D_IN, _D_HID)), _part_spec(16)],
        out_specs=[_row_spec(_D_HID), _row_spec(_D_HID)],
        out_shape=[jax.ShapeDtypeStruct((_N, _D_HID), jnp.float32)] * 2,
    )(x, W1, degp)

    p1 = _make_agg(_D_HID)(s1, src_p, dst_p)

    h2, s2 = pl.pallas_call(
        _tc2_body,
        grid=(_GRID,),
        in_specs=[_part_spec(16), _part_spec(_D_HID), _row_spec(_D_HID),
                  _full_spec((1, _D_HID)), _full_spec((_D_HID, _D_OUT))],
        out_specs=[_row_spec(_D_OUT), _row_spec(_D_OUT)],
        out_shape=[jax.ShapeDtypeStruct((_N, _D_OUT), jnp.float32)] * 2,
    )(degp, p1, h1, b1.reshape(1, _D_HID), W2)

    p2 = _make_agg(_D_OUT)(s2, src_p, dst_p)

    out = pl.pallas_call(
        _tc3_body,
        grid=(_GRID,),
        in_specs=[_part_spec(16), _part_spec(_D_OUT), _row_spec(_D_OUT),
                  _full_spec((1, _D_OUT))],
        out_specs=_row_spec(_D_OUT),
        out_shape=jax.ShapeDtypeStruct((_N, _D_OUT), jnp.float32),
    )(degp, p2, h2, b2.reshape(1, _D_OUT))

    return out


# speedup vs baseline: 21.4123x; 1.0029x over previous
import functools

import jax
import jax.numpy as jnp
from jax import lax
from jax.experimental import pallas as pl
from jax.experimental.pallas import tpu as pltpu
from jax.experimental.pallas import tpu_sc as plsc

_N = 10000
_E = 320000
_D_IN, _D_HID, _D_OUT = 128, 64, 32

_NC, _NS = 2, 16
_NW = _NC * _NS
_C = 128
_K = 80
_EPAD = _NW * _K * _C
_ACC = 10240
_RPT = _ACC // _NS
_DUMP = _N

_BN = 2000


def _mesh():
    return plsc.VectorSubcoreMesh(core_axis_name="c", subcore_axis_name="s")


@functools.partial(
    pl.kernel,
    out_type=jax.ShapeDtypeStruct((_NC, _ACC, 16), jnp.float32),
    mesh=_mesh(),
    compiler_params=pltpu.CompilerParams(use_tc_tiling_on_sc=False),
    scratch_types=[
        pltpu.VMEM((_K, _C), jnp.int32),
        pltpu.VMEM((_C, 16), jnp.float32),
        pltpu.VMEM((_RPT, 16), jnp.float32),
        pltpu.VMEM_SHARED((_ACC, 16), jnp.float32),
    ],
)
def _deg_kernel(dst_hbm, out_hbm, dst_v, ones_v, stage_v, acc_sh):
    cid = lax.axis_index("c")
    sid = lax.axis_index("s")
    wid = sid * _NC + cid

    def fill_ones(i, c):
        ones_v[i, :] = jnp.ones((16,), jnp.float32)
        return c

    lax.fori_loop(0, _C, fill_ones, 0)

    def fill_zero(i, c):
        stage_v[i, :] = jnp.zeros((16,), jnp.float32)
        return c

    lax.fori_loop(0, _RPT, fill_zero, 0)

    pltpu.sync_copy(stage_v, acc_sh.at[pl.ds(sid * _RPT, _RPT)])
    plsc.subcore_barrier()

    pltpu.sync_copy(dst_hbm.at[wid], dst_v)

    def chunk(j, c):
        pltpu.sync_copy(ones_v, acc_sh.at[dst_v.at[j]], add=True)
        return c

    lax.fori_loop(0, _K, chunk, 0)

    plsc.subcore_barrier()
    pltpu.sync_copy(acc_sh.at[pl.ds(sid * _RPT, _RPT)], stage_v)
    pltpu.sync_copy(stage_v, out_hbm.at[cid, pl.ds(sid * _RPT, _RPT)])


def _make_agg(D):
    @functools.partial(
        pl.kernel,
        out_type=jax.ShapeDtypeStruct((_NC, _ACC, D), jnp.float32),
        mesh=_mesh(),
        compiler_params=pltpu.CompilerParams(use_tc_tiling_on_sc=False),
        scratch_types=[
            pltpu.VMEM((_K, _C), jnp.int32),
            pltpu.VMEM((_K, _C), jnp.int32),
            pltpu.VMEM((_C, D), jnp.float32),
            pltpu.VMEM((_C, D), jnp.float32),
            pltpu.VMEM((_C, D), jnp.float32),
            pltpu.VMEM((_C, D), jnp.float32),
            pltpu.VMEM((_C, D), jnp.float32),
            pltpu.VMEM((_C, D), jnp.float32),
            pltpu.VMEM_SHARED((_ACC, D), jnp.float32),
            pltpu.SemaphoreType.DMA,
            pltpu.SemaphoreType.DMA,
            pltpu.SemaphoreType.DMA,
            pltpu.SemaphoreType.DMA,
            pltpu.SemaphoreType.DMA,
        ],
    )
    def _agg(s_hbm, src_hbm, dst_hbm, out_hbm,
             src_v, dst_v, r0, r1, r2, r3, r4, zb, acc_sh,
             g0, g1, g2, g3, g4):
        cid = lax.axis_index("c")
        sid = lax.axis_index("s")
        wid = sid * _NC + cid
        nz = D // 16

        def fill_zero(i, c):
            for l in range(nz):
                zb[i, pl.ds(l * 16, 16)] = jnp.zeros((16,), jnp.float32)
            return c

        lax.fori_loop(0, _C, fill_zero, 0)

        for q in range(_RPT // _C):
            pltpu.sync_copy(zb, acc_sh.at[pl.ds(sid * _RPT + q * _C, _C)])
        plsc.subcore_barrier()

        pltpu.sync_copy(src_hbm.at[wid], src_v)
        pltpu.sync_copy(dst_hbm.at[wid], dst_v)

        bufs = (r0, r1, r2, r3, r4)
        sems = (g0, g1, g2, g3, g4)
        nb = 5

        for l in range(nb):
            pltpu.async_copy(s_hbm.at[src_v.at[l]], bufs[l], sems[l])

        def step(j4, c):
            for l in range(nb):
                jc = j4 * nb + l
                pltpu.make_async_copy(
                    s_hbm.at[src_v.at[jc]], bufs[l], sems[l]).wait()
                pltpu.sync_copy(bufs[l], acc_sh.at[dst_v.at[jc]], add=True)
                pltpu.async_copy(
                    s_hbm.at[src_v.at[jc + nb]], bufs[l], sems[l])
            return c

        lax.fori_loop(0, _K // nb - 1, step, 0)

        for l in range(nb):
            jc = _K - nb + l
            pltpu.make_async_copy(
                s_hbm.at[src_v.at[jc]], bufs[l], sems[l]).wait()
            pltpu.sync_copy(bufs[l], acc_sh.at[dst_v.at[jc]], add=True)

        plsc.subcore_barrier()
        for q in range(_RPT // _C):
            pltpu.sync_copy(acc_sh.at[pl.ds(sid * _RPT + q * _C, _C)], zb)
            pltpu.sync_copy(zb, out_hbm.at[cid, pl.ds(sid * _RPT + q * _C, _C)])

    return _agg


def _tc1_body(x_ref, w1_ref, degp_ref, h1_ref, s1_ref):
    deg = degp_ref[0, :, 0:1] + degp_ref[1, :, 0:1] + 1.0
    dis = lax.rsqrt(deg)
    h = jnp.dot(x_ref[...], w1_ref[...], preferred_element_type=jnp.float32)
    h1_ref[...] = h
    s1_ref[...] = h * dis


def _tc2_body(degp_ref, p1_ref, h1_ref, b1_ref, w2_ref, h2_ref, s2_ref):
    deg = degp_ref[0, :, 0:1] + degp_ref[1, :, 0:1] + 1.0
    dis = lax.rsqrt(deg)
    agg = p1_ref[0] + p1_ref[1]
    x1 = dis * agg + (dis * dis) * h1_ref[...] + b1_ref[...]
    r = jnp.maximum(x1, 0.0)
    h2 = jnp.dot(r, w2_ref[...], preferred_element_type=jnp.float32)
    h2_ref[...] = h2
    s2_ref[...] = h2 * dis


def _tc3_body(degp_ref, p2_ref, h2_ref, b2_ref, out_ref):
    deg = degp_ref[0, :, 0:1] + degp_ref[1, :, 0:1] + 1.0
    dis = lax.rsqrt(deg)
    agg = p2_ref[0] + p2_ref[1]
    out_ref[...] = dis * agg + (dis * dis) * h2_ref[...] + b2_ref[...]


def _row_spec(d):
    return pl.BlockSpec((_BN, d), lambda i: (i, 0))


def _part_spec(d):
    return pl.BlockSpec((2, _BN, d), lambda i: (0, i, 0))


def _full_spec(shape):
    return pl.BlockSpec(shape, lambda i: tuple(0 for _ in shape))


_GRID = _N // _BN


def kernel(x, edge_index, W1, b1, W2, b2):
    src = edge_index[0].astype(jnp.int32)
    dst = edge_index[1].astype(jnp.int32)
    pad = _EPAD - _E
    src_p = jnp.concatenate([src, jnp.zeros((pad,), jnp.int32)]).reshape(_NW, _K, _C)
    dst_p = jnp.concatenate([dst, jnp.full((pad,), _DUMP, jnp.int32)]).reshape(_NW, _K, _C)

    degp = _deg_kernel(dst_p)

    h1, s1 = pl.pallas_call(
        _tc1_body,
        grid=(_GRID,),
        in_specs=[_row_spec(_D_IN), _full_spec((_D_IN, _D_HID)), _part_spec(16)],
        out_specs=[_row_spec(_D_HID), _row_spec(_D_HID)],
        out_shape=[jax.ShapeDtypeStruct((_N, _D_HID), jnp.float32)] * 2,
    )(x, W1, degp)

    p1 = _make_agg(_D_HID)(s1, src_p, dst_p)

    h2, s2 = pl.pallas_call(
        _tc2_body,
        grid=(_GRID,),
        in_specs=[_part_spec(16), _part_spec(_D_HID), _row_spec(_D_HID),
                  _full_spec((1, _D_HID)), _full_spec((_D_HID, _D_OUT))],
        out_specs=[_row_spec(_D_OUT), _row_spec(_D_OUT)],
        out_shape=[jax.ShapeDtypeStruct((_N, _D_OUT), jnp.float32)] * 2,
    )(degp, p1, h1, b1.reshape(1, _D_HID), W2)

    p2 = _make_agg(_D_OUT)(s2, src_p, dst_p)

    out = pl.pallas_call(
        _tc3_body,
        grid=(_GRID,),
        in_specs=[_part_spec(16), _part_spec(_D_OUT), _row_spec(_D_OUT),
                  _full_spec((1, _D_OUT))],
        out_specs=_row_spec(_D_OUT),
        out_shape=jax.ShapeDtypeStruct((_N, _D_OUT), jnp.float32),
    )(degp, p2, h2, b2.reshape(1, _D_OUT))

    return out


# speedup vs baseline: 21.4224x; 1.0005x over previous
import functools

import jax
import jax.numpy as jnp
from jax import lax
from jax.experimental import pallas as pl
from jax.experimental.pallas import tpu as pltpu
from jax.experimental.pallas import tpu_sc as plsc

_N = 10000
_E = 320000
_D_IN, _D_HID, _D_OUT = 128, 64, 32

_NC, _NS = 2, 16
_NW = _NC * _NS
_C = 128
_K = 80
_EPAD = _NW * _K * _C
_ACC = 10240
_RPT = _ACC // _NS
_DUMP = _N

_BN = 2000


def _mesh():
    return plsc.VectorSubcoreMesh(core_axis_name="c", subcore_axis_name="s")


@functools.partial(
    pl.kernel,
    out_type=jax.ShapeDtypeStruct((_NC, _ACC, 16), jnp.float32),
    mesh=_mesh(),
    compiler_params=pltpu.CompilerParams(use_tc_tiling_on_sc=False),
    scratch_types=[
        pltpu.VMEM((_K, _C), jnp.int32),
        pltpu.VMEM((_C, 16), jnp.float32),
        pltpu.VMEM((_RPT, 16), jnp.float32),
        pltpu.VMEM_SHARED((_ACC, 16), jnp.float32),
    ],
)
def _deg_kernel(dst_hbm, out_hbm, dst_v, ones_v, stage_v, acc_sh):
    cid = lax.axis_index("c")
    sid = lax.axis_index("s")
    wid = sid * _NC + cid

    def fill_ones(i, c):
        ones_v[i, :] = jnp.ones((16,), jnp.float32)
        return c

    lax.fori_loop(0, _C, fill_ones, 0)

    def fill_zero(i, c):
        stage_v[i, :] = jnp.zeros((16,), jnp.float32)
        return c

    lax.fori_loop(0, _RPT, fill_zero, 0)

    pltpu.sync_copy(stage_v, acc_sh.at[pl.ds(sid * _RPT, _RPT)])
    plsc.subcore_barrier()

    pltpu.sync_copy(dst_hbm.at[wid], dst_v)

    def chunk(j, c):
        pltpu.sync_copy(ones_v, acc_sh.at[dst_v.at[j]], add=True)
        return c

    lax.fori_loop(0, _K, chunk, 0)

    plsc.subcore_barrier()
    pltpu.sync_copy(acc_sh.at[pl.ds(sid * _RPT, _RPT)], stage_v)
    pltpu.sync_copy(stage_v, out_hbm.at[cid, pl.ds(sid * _RPT, _RPT)])


def _make_agg(D):
    @functools.partial(
        pl.kernel,
        out_type=jax.ShapeDtypeStruct((_NC, _ACC, D), jnp.float32),
        mesh=_mesh(),
        compiler_params=pltpu.CompilerParams(use_tc_tiling_on_sc=False),
        scratch_types=[
            pltpu.VMEM((_K, _C), jnp.int32),
            pltpu.VMEM((_K, _C), jnp.int32),
            pltpu.VMEM((_C, D), jnp.float32),
            pltpu.VMEM((_C, D), jnp.float32),
            pltpu.VMEM((_C, D), jnp.float32),
            pltpu.VMEM((_C, D), jnp.float32),
            pltpu.VMEM((_C, D), jnp.float32),
            pltpu.VMEM((_C, D), jnp.float32),
            pltpu.VMEM_SHARED((_ACC, D), jnp.float32),
            pltpu.SemaphoreType.DMA,
            pltpu.SemaphoreType.DMA,
            pltpu.SemaphoreType.DMA,
            pltpu.SemaphoreType.DMA,
            pltpu.SemaphoreType.DMA,
        ],
    )
    def _agg(s_hbm, src_hbm, dst_hbm, out_hbm,
             src_v, dst_v, r0, r1, r2, r3, r4, zb, acc_sh,
             g0, g1, g2, g3, g4):
        cid = lax.axis_index("c")
        sid = lax.axis_index("s")
        wid = sid * _NC + cid
        nz = D // 16

        def fill_zero(i, c):
            for l in range(nz):
                zb[i, pl.ds(l * 16, 16)] = jnp.zeros((16,), jnp.float32)
            return c

        lax.fori_loop(0, _C, fill_zero, 0)

        for q in range(_RPT // _C):
            pltpu.sync_copy(zb, acc_sh.at[pl.ds(sid * _RPT + q * _C, _C)])
        plsc.subcore_barrier()

        pltpu.sync_copy(src_hbm.at[wid], src_v)
        pltpu.sync_copy(dst_hbm.at[wid], dst_v)

        bufs = (r0, r1, r2, r3, r4)
        gsem = (g0, g1, g2, g3, g4)
        nb = 5

        for l in range(nb):
            pltpu.async_copy(s_hbm.at[src_v.at[l]], bufs[l], gsem[l])

        def step(j4, c):
            for l in range(nb):
                jc = j4 * nb + l
                pltpu.make_async_copy(
                    s_hbm.at[src_v.at[jc]], bufs[l], gsem[l]).wait()
                pltpu.sync_copy(bufs[l], acc_sh.at[dst_v.at[jc]], add=True)
                pltpu.async_copy(
                    s_hbm.at[src_v.at[jc + nb]], bufs[l], gsem[l])
            return c

        lax.fori_loop(0, _K // nb - 1, step, 0)

        for l in range(nb):
            jc = _K - nb + l
            pltpu.make_async_copy(
                s_hbm.at[src_v.at[jc]], bufs[l], gsem[l]).wait()
            pltpu.sync_copy(bufs[l], acc_sh.at[dst_v.at[jc]], add=True)

        plsc.subcore_barrier()
        for q in range(_RPT // _C):
            pltpu.sync_copy(acc_sh.at[pl.ds(sid * _RPT + q * _C, _C)], zb)
            pltpu.sync_copy(zb, out_hbm.at[cid, pl.ds(sid * _RPT + q * _C, _C)])

    return _agg


def _tc1_body(x_ref, w1_ref, degp_ref, h1_ref, s1_ref):
    deg = degp_ref[0, :, 0:1] + degp_ref[1, :, 0:1] + 1.0
    dis = lax.rsqrt(deg)
    h = jnp.dot(x_ref[...], w1_ref[...], preferred_element_type=jnp.float32)
    h1_ref[...] = h
    s1_ref[...] = h * dis


def _tc2_body(degp_ref, p1_ref, h1_ref, b1_ref, w2_ref, h2_ref, s2_ref):
    deg = degp_ref[0, :, 0:1] + degp_ref[1, :, 0:1] + 1.0
    dis = lax.rsqrt(deg)
    agg = p1_ref[0] + p1_ref[1]
    x1 = dis * agg + (dis * dis) * h1_ref[...] + b1_ref[...]
    r = jnp.maximum(x1, 0.0)
    h2 = jnp.dot(r, w2_ref[...], preferred_element_type=jnp.float32)
    h2_ref[...] = h2
    s2_ref[...] = h2 * dis


def _tc3_body(degp_ref, p2_ref, h2_ref, b2_ref, out_ref):
    deg = degp_ref[0, :, 0:1] + degp_ref[1, :, 0:1] + 1.0
    dis = lax.rsqrt(deg)
    agg = p2_ref[0] + p2_ref[1]
    out_ref[...] = dis * agg + (dis * dis) * h2_ref[...] + b2_ref[...]


def _row_spec(d):
    return pl.BlockSpec((_BN, d), lambda i: (i, 0))


def _part_spec(d):
    return pl.BlockSpec((2, _BN, d), lambda i: (0, i, 0))


def _full_spec(shape):
    return pl.BlockSpec(shape, lambda i: tuple(0 for _ in shape))


_GRID = _N // _BN


def kernel(x, edge_index, W1, b1, W2, b2):
    src = edge_index[0].astype(jnp.int32)
    dst = edge_index[1].astype(jnp.int32)
    pad = _EPAD - _E
    src_p = jnp.concatenate([src, jnp.zeros((pad,), jnp.int32)]).reshape(_NW, _K, _C)
    dst_p = jnp.concatenate([dst, jnp.full((pad,), _DUMP, jnp.int32)]).reshape(_NW, _K, _C)

    degp = _deg_kernel(dst_p)

    h1, s1 = pl.pallas_call(
        _tc1_body,
        grid=(_GRID,),
        in_specs=[_row_spec(_D_IN), _full_spec((_D_IN, _D_HID)), _part_spec(16)],
        out_specs=[_row_spec(_D_HID), _row_spec(_D_HID)],
        out_shape=[jax.ShapeDtypeStruct((_N, _D_HID), jnp.float32)] * 2,
    )(x, W1, degp)

    p1 = _make_agg(_D_HID)(s1, src_p, dst_p)

    h2, s2 = pl.pallas_call(
        _tc2_body,
        grid=(_GRID,),
        in_specs=[_part_spec(16), _part_spec(_D_HID), _row_spec(_D_HID),
                  _full_spec((1, _D_HID)), _full_spec((_D_HID, _D_OUT))],
        out_specs=[_row_spec(_D_OUT), _row_spec(_D_OUT)],
        out_shape=[jax.ShapeDtypeStruct((_N, _D_OUT), jnp.float32)] * 2,
    )(degp, p1, h1, b1.reshape(1, _D_HID), W2)

    p2 = _make_agg(_D_OUT)(s2, src_p, dst_p)

    out = pl.pallas_call(
        _tc3_body,
        grid=(_GRID,),
        in_specs=[_part_spec(16), _part_spec(_D_OUT), _row_spec(_D_OUT),
                  _full_spec((1, _D_OUT))],
        out_specs=_row_spec(_D_OUT),
        out_shape=jax.ShapeDtypeStruct((_N, _D_OUT), jnp.float32),
    )(degp, p2, h2, b2.reshape(1, _D_OUT))

    return out
